# baseline (speedup 1.0000x reference)
import functools

import jax
import jax.numpy as jnp
from jax import lax
from jax.experimental import pallas as pl
from jax.experimental.pallas import tpu as pltpu
from jax.experimental.pallas import tpu_sc as plsc

HIDDEN = 128
EMSIZE = 768
CHUNK = 80


def _sc_gather(table, idx, off, ns):
    d = table.shape[1]
    mesh = plsc.VectorSubcoreMesh(core_axis_name="c", subcore_axis_name="s")
    num_cores = mesh.num_cores
    nw = num_cores * mesh.num_subcores
    per_w = ns // nw
    n_ch = per_w // CHUNK

    @functools.partial(
        pl.kernel,
        mesh=mesh,
        out_type=jax.ShapeDtypeStruct((ns, d), jnp.float32),
        scratch_types=[
            pltpu.VMEM((per_w,), jnp.int32),
            pltpu.VMEM((2, CHUNK, d), jnp.float32),
            pltpu.SemaphoreType.DMA,
            pltpu.SemaphoreType.DMA,
        ],
    )
    def gather_kernel(table_hbm, idx_hbm, out_hbm, idx_v, rows_v, sem0, sem1):
        wid = lax.axis_index("s") * num_cores + lax.axis_index("c")
        base = wid * per_w
        pltpu.sync_copy(idx_hbm.at[pl.ds(off + base, per_w)], idx_v)
        sems = (sem0, sem1)

        def fire(c):
            return pltpu.async_copy(
                table_hbm.at[idx_v.at[pl.ds(c * CHUNK, CHUNK)]],
                rows_v.at[c % 2],
                sems[c % 2],
            )

        cp = fire(0)
        for c in range(1, n_ch):
            nxt = fire(c)
            cp.wait()
            pltpu.sync_copy(
                rows_v.at[(c - 1) % 2],
                out_hbm.at[pl.ds(base + (c - 1) * CHUNK, CHUNK)],
            )
            cp = nxt
        cp.wait()
        pltpu.sync_copy(
            rows_v.at[(n_ch - 1) % 2],
            out_hbm.at[pl.ds(base + (n_ch - 1) * CHUNK, CHUNK)],
        )

    return gather_kernel(table, idx)


def _tc_project_slice(acc, x, w, b, row0, n):
    ns = x.shape[0]
    bm = 6400
    assert ns % bm == 0 and row0 % bm == 0
    blk0 = row0 // bm

    def body(*refs):
        x_ref, w_ref, b_ref, o_ref = refs[-4:]
        o_ref[...] = (
            jnp.dot(x_ref[...], w_ref[...], preferred_element_type=jnp.float32)
            + b_ref[...]
        )

    in_specs = [
        pl.BlockSpec((bm, HIDDEN), lambda i: (i, 0)),
        pl.BlockSpec((HIDDEN, EMSIZE), lambda i: (0, 0)),
        pl.BlockSpec((1, EMSIZE), lambda i: (0, 0)),
    ]
    args = (x, w, b.reshape(1, EMSIZE))
    aliases = {}
    if acc is not None:
        in_specs = [pl.BlockSpec(memory_space=pl.ANY)] + in_specs
        args = (acc,) + args
        aliases = {0: 0}
    return pl.pallas_call(
        body,
        grid=(ns // bm,),
        in_specs=in_specs,
        out_specs=pl.BlockSpec((bm, EMSIZE), lambda i: (i + blk0, 0)),
        out_shape=jax.ShapeDtypeStruct((n, EMSIZE), jnp.float32),
        input_output_aliases=aliases,
    )(*args)


def kernel(item_embeds, emb_table, W_proj, b_proj):
    batch, hist = item_embeds.shape
    n = batch * hist
    idx = item_embeds.T.reshape(-1)
    n_slices = 1
    ns = n // n_slices
    rows = [_sc_gather(emb_table, idx, s * ns, ns) for s in range(n_slices)]
    out = None
    for s in range(n_slices):
        out = _tc_project_slice(out, rows[s], W_proj, b_proj, s * ns, n)
    return out.reshape(hist, batch, EMSIZE).transpose(1, 0, 2)


# speedup vs baseline: 1.0033x; 1.0033x over previous
import functools

import jax
import jax.numpy as jnp
from jax import lax
from jax.experimental import pallas as pl
from jax.experimental.pallas import tpu as pltpu
from jax.experimental.pallas import tpu_sc as plsc

HIDDEN = 128
EMSIZE = 768
CHUNK = 80


def _sc_gather(table, idx, off, ns):
    d = table.shape[1]
    mesh = plsc.VectorSubcoreMesh(core_axis_name="c", subcore_axis_name="s")
    num_cores = mesh.num_cores
    nw = num_cores * mesh.num_subcores
    per_w = ns // nw
    n_ch = per_w // CHUNK

    @functools.partial(
        pl.kernel,
        mesh=mesh,
        out_type=jax.ShapeDtypeStruct((ns, d), jnp.float32),
        scratch_types=[
            pltpu.VMEM((per_w,), jnp.int32),
            pltpu.VMEM((2, CHUNK, d), jnp.float32),
            pltpu.SemaphoreType.DMA,
            pltpu.SemaphoreType.DMA,
        ],
    )
    def gather_kernel(table_hbm, idx_hbm, out_hbm, idx_v, rows_v, sem0, sem1):
        wid = lax.axis_index("s") * num_cores + lax.axis_index("c")
        base = wid * per_w
        pltpu.sync_copy(idx_hbm.at[pl.ds(off + base, per_w)], idx_v)
        sems = (sem0, sem1)

        def fire(c):
            return pltpu.async_copy(
                table_hbm.at[idx_v.at[pl.ds(c * CHUNK, CHUNK)]],
                rows_v.at[c % 2],
                sems[c % 2],
            )

        cp = fire(0)
        for c in range(1, n_ch):
            nxt = fire(c)
            cp.wait()
            pltpu.sync_copy(
                rows_v.at[(c - 1) % 2],
                out_hbm.at[pl.ds(base + (c - 1) * CHUNK, CHUNK)],
            )
            cp = nxt
        cp.wait()
        pltpu.sync_copy(
            rows_v.at[(n_ch - 1) % 2],
            out_hbm.at[pl.ds(base + (n_ch - 1) * CHUNK, CHUNK)],
        )

    return gather_kernel(table, idx)


def _tc_project_slice(acc, x, w, b, row0, n):
    ns = x.shape[0]
    bm = 5120
    assert ns % bm == 0 and row0 % bm == 0
    blk0 = row0 // bm

    def body(*refs):
        x_ref, w_ref, b_ref, o_ref = refs[-4:]
        o_ref[...] = (
            jnp.dot(x_ref[...], w_ref[...], preferred_element_type=jnp.float32)
            + b_ref[...]
        )

    in_specs = [
        pl.BlockSpec((bm, HIDDEN), lambda i: (i, 0)),
        pl.BlockSpec((HIDDEN, EMSIZE), lambda i: (0, 0)),
        pl.BlockSpec((1, EMSIZE), lambda i: (0, 0)),
    ]
    args = (x, w, b.reshape(1, EMSIZE))
    aliases = {}
    if acc is not None:
        in_specs = [pl.BlockSpec(memory_space=pl.ANY)] + in_specs
        args = (acc,) + args
        aliases = {0: 0}
    return pl.pallas_call(
        body,
        grid=(ns // bm,),
        in_specs=in_specs,
        out_specs=pl.BlockSpec((bm, EMSIZE), lambda i: (i + blk0, 0)),
        out_shape=jax.ShapeDtypeStruct((n, EMSIZE), jnp.float32),
        input_output_aliases=aliases,
    )(*args)


def kernel(item_embeds, emb_table, W_proj, b_proj):
    batch, hist = item_embeds.shape
    n = batch * hist
    idx = item_embeds.T.reshape(-1)
    n_slices = 1
    ns = n // n_slices
    rows = [_sc_gather(emb_table, idx, s * ns, ns) for s in range(n_slices)]
    out = None
    for s in range(n_slices):
        out = _tc_project_slice(out, rows[s], W_proj, b_proj, s * ns, n)
    return out.reshape(hist, batch, EMSIZE).transpose(1, 0, 2)


# speedup vs baseline: 1.0324x; 1.0289x over previous
import functools

import jax
import jax.numpy as jnp
from jax import lax
from jax.experimental import pallas as pl
from jax.experimental.pallas import tpu as pltpu
from jax.experimental.pallas import tpu_sc as plsc

HIDDEN = 128
EMSIZE = 768
CHUNK = 80


def _sc_gather(table, idx, off, ns):
    d = table.shape[1]
    mesh = plsc.VectorSubcoreMesh(core_axis_name="c", subcore_axis_name="s")
    num_cores = mesh.num_cores
    nw = num_cores * mesh.num_subcores
    per_w = ns // nw
    sub = 4
    mega = sub * CHUNK
    n_mega = per_w // mega

    @functools.partial(
        pl.kernel,
        mesh=mesh,
        out_type=jax.ShapeDtypeStruct((ns, d), jnp.float32),
        scratch_types=[
            pltpu.VMEM((per_w,), jnp.int32),
            pltpu.VMEM((2, mega, d), jnp.float32),
            pltpu.SemaphoreType.DMA,
            pltpu.SemaphoreType.DMA,
        ],
    )
    def gather_kernel(table_hbm, idx_hbm, out_hbm, idx_v, rows_v, sem0, sem1):
        wid = lax.axis_index("s") * num_cores + lax.axis_index("c")
        base = wid * per_w
        pltpu.sync_copy(idx_hbm.at[pl.ds(off + base, per_w)], idx_v)
        sems = (sem0, sem1)

        def fire(m, buf):
            return [
                pltpu.async_copy(
                    table_hbm.at[
                        idx_v.at[pl.ds(m * mega + r * CHUNK, CHUNK)]
                    ],
                    rows_v.at[buf, pl.ds(r * CHUNK, CHUNK)],
                    sems[buf],
                )
                for r in range(sub)
            ]

        cps = fire(0, 0)
        for m in range(1, n_mega):
            nxt = fire(m, m % 2)
            for cp in cps:
                cp.wait()
            pltpu.sync_copy(
                rows_v.at[(m - 1) % 2],
                out_hbm.at[pl.ds(base + (m - 1) * mega, mega)],
            )
            cps = nxt
        for cp in cps:
            cp.wait()
        pltpu.sync_copy(
            rows_v.at[(n_mega - 1) % 2],
            out_hbm.at[pl.ds(base + (n_mega - 1) * mega, mega)],
        )

    return gather_kernel(table, idx)


def _tc_project_slice(acc, x, w, b, row0, n):
    ns = x.shape[0]
    bm = 5120
    assert ns % bm == 0 and row0 % bm == 0
    blk0 = row0 // bm

    def body(*refs):
        x_ref, w_ref, b_ref, o_ref = refs[-4:]
        o_ref[...] = (
            jnp.dot(x_ref[...], w_ref[...], preferred_element_type=jnp.float32)
            + b_ref[...]
        )

    in_specs = [
        pl.BlockSpec((bm, HIDDEN), lambda i: (i, 0)),
        pl.BlockSpec((HIDDEN, EMSIZE), lambda i: (0, 0)),
        pl.BlockSpec((1, EMSIZE), lambda i: (0, 0)),
    ]
    args = (x, w, b.reshape(1, EMSIZE))
    aliases = {}
    if acc is not None:
        in_specs = [pl.BlockSpec(memory_space=pl.ANY)] + in_specs
        args = (acc,) + args
        aliases = {0: 0}
    return pl.pallas_call(
        body,
        grid=(ns // bm,),
        in_specs=in_specs,
        out_specs=pl.BlockSpec((bm, EMSIZE), lambda i: (i + blk0, 0)),
        out_shape=jax.ShapeDtypeStruct((n, EMSIZE), jnp.float32),
        input_output_aliases=aliases,
    )(*args)


def kernel(item_embeds, emb_table, W_proj, b_proj):
    batch, hist = item_embeds.shape
    n = batch * hist
    idx = item_embeds.T.reshape(-1)
    n_slices = 1
    ns = n // n_slices
    rows = [_sc_gather(emb_table, idx, s * ns, ns) for s in range(n_slices)]
    out = None
    for s in range(n_slices):
        out = _tc_project_slice(out, rows[s], W_proj, b_proj, s * ns, n)
    return out.reshape(hist, batch, EMSIZE).transpose(1, 0, 2)


# speedup vs baseline: 1.0336x; 1.0012x over previous
import functools

import jax
import jax.numpy as jnp
from jax import lax
from jax.experimental import pallas as pl
from jax.experimental.pallas import tpu as pltpu
from jax.experimental.pallas import tpu_sc as plsc

HIDDEN = 128
EMSIZE = 768
CHUNK = 80


def _sc_gather(table, idx, off, ns):
    d = table.shape[1]
    mesh = plsc.VectorSubcoreMesh(core_axis_name="c", subcore_axis_name="s")
    num_cores = mesh.num_cores
    nw = num_cores * mesh.num_subcores
    per_w = ns // nw
    sub = 5
    mega = sub * CHUNK
    n_mega = per_w // mega

    @functools.partial(
        pl.kernel,
        mesh=mesh,
        out_type=jax.ShapeDtypeStruct((ns, d), jnp.float32),
        scratch_types=[
            pltpu.VMEM((per_w,), jnp.int32),
            pltpu.VMEM((2, mega, d), jnp.float32),
            pltpu.SemaphoreType.DMA,
            pltpu.SemaphoreType.DMA,
        ],
    )
    def gather_kernel(table_hbm, idx_hbm, out_hbm, idx_v, rows_v, sem0, sem1):
        wid = lax.axis_index("s") * num_cores + lax.axis_index("c")
        base = wid * per_w
        pltpu.sync_copy(idx_hbm.at[pl.ds(off + base, per_w)], idx_v)
        sems = (sem0, sem1)

        def fire(m, buf):
            return [
                pltpu.async_copy(
                    table_hbm.at[
                        idx_v.at[pl.ds(m * mega + r * CHUNK, CHUNK)]
                    ],
                    rows_v.at[buf, pl.ds(r * CHUNK, CHUNK)],
                    sems[buf],
                )
                for r in range(sub)
            ]

        cps = fire(0, 0)
        for m in range(1, n_mega):
            nxt = fire(m, m % 2)
            for cp in cps:
                cp.wait()
            pltpu.sync_copy(
                rows_v.at[(m - 1) % 2],
                out_hbm.at[pl.ds(base + (m - 1) * mega, mega)],
            )
            cps = nxt
        for cp in cps:
            cp.wait()
        pltpu.sync_copy(
            rows_v.at[(n_mega - 1) % 2],
            out_hbm.at[pl.ds(base + (n_mega - 1) * mega, mega)],
        )

    return gather_kernel(table, idx)


def _tc_project_slice(acc, x, w, b, row0, n):
    ns = x.shape[0]
    bm = 5120
    assert ns % bm == 0 and row0 % bm == 0
    blk0 = row0 // bm

    def body(*refs):
        x_ref, w_ref, b_ref, o_ref = refs[-4:]
        o_ref[...] = (
            jnp.dot(x_ref[...], w_ref[...], preferred_element_type=jnp.float32)
            + b_ref[...]
        )

    in_specs = [
        pl.BlockSpec((bm, HIDDEN), lambda i: (i, 0)),
        pl.BlockSpec((HIDDEN, EMSIZE), lambda i: (0, 0)),
        pl.BlockSpec((1, EMSIZE), lambda i: (0, 0)),
    ]
    args = (x, w, b.reshape(1, EMSIZE))
    aliases = {}
    if acc is not None:
        in_specs = [pl.BlockSpec(memory_space=pl.ANY)] + in_specs
        args = (acc,) + args
        aliases = {0: 0}
    return pl.pallas_call(
        body,
        grid=(ns // bm,),
        in_specs=in_specs,
        out_specs=pl.BlockSpec((bm, EMSIZE), lambda i: (i + blk0, 0)),
        out_shape=jax.ShapeDtypeStruct((n, EMSIZE), jnp.float32),
        input_output_aliases=aliases,
    )(*args)


def kernel(item_embeds, emb_table, W_proj, b_proj):
    batch, hist = item_embeds.shape
    n = batch * hist
    idx = item_embeds.T.reshape(-1)
    n_slices = 1
    ns = n // n_slices
    rows = [_sc_gather(emb_table, idx, s * ns, ns) for s in range(n_slices)]
    out = None
    for s in range(n_slices):
        out = _tc_project_slice(out, rows[s], W_proj, b_proj, s * ns, n)
    return out.reshape(hist, batch, EMSIZE).transpose(1, 0, 2)


# speedup vs baseline: 1.0342x; 1.0006x over previous
import functools

import jax
import jax.numpy as jnp
from jax import lax
from jax.experimental import pallas as pl
from jax.experimental.pallas import tpu as pltpu
from jax.experimental.pallas import tpu_sc as plsc

HIDDEN = 128
EMSIZE = 768
CHUNK = 80
SUB = 5
BM = 5120


def _sc_gather(table, idx):
    n = idx.shape[0]
    d = table.shape[1]
    mesh = plsc.VectorSubcoreMesh(core_axis_name="c", subcore_axis_name="s")
    num_cores = mesh.num_cores
    nw = num_cores * mesh.num_subcores
    per_w = n // nw
    mega = SUB * CHUNK
    n_mega = per_w // mega

    @functools.partial(
        pl.kernel,
        mesh=mesh,
        out_type=jax.ShapeDtypeStruct((n, d), jnp.float32),
        scratch_types=[
            pltpu.VMEM((per_w,), jnp.int32),
            pltpu.VMEM((2, mega, d), jnp.float32),
            pltpu.SemaphoreType.DMA,
            pltpu.SemaphoreType.DMA,
        ],
    )
    def gather_kernel(table_hbm, idx_hbm, out_hbm, idx_v, rows_v, sem0, sem1):
        wid = lax.axis_index("s") * num_cores + lax.axis_index("c")
        base = wid * per_w
        pltpu.sync_copy(idx_hbm.at[pl.ds(base, per_w)], idx_v)
        sems = (sem0, sem1)

        def fire(m, buf):
            return [
                pltpu.async_copy(
                    table_hbm.at[
                        idx_v.at[pl.ds(m * mega + r * CHUNK, CHUNK)]
                    ],
                    rows_v.at[buf, pl.ds(r * CHUNK, CHUNK)],
                    sems[buf],
                )
                for r in range(SUB)
            ]

        cps = fire(0, 0)
        for m in range(1, n_mega):
            nxt = fire(m, m % 2)
            for cp in cps:
                cp.wait()
            pltpu.sync_copy(
                rows_v.at[(m - 1) % 2],
                out_hbm.at[pl.ds(base + (m - 1) * mega, mega)],
            )
            cps = nxt
        for cp in cps:
            cp.wait()
        pltpu.sync_copy(
            rows_v.at[(n_mega - 1) % 2],
            out_hbm.at[pl.ds(base + (n_mega - 1) * mega, mega)],
        )

    return gather_kernel(table, idx)


def _tc_project(x, w, b):
    n = x.shape[0]

    def body(x_ref, w_ref, b_ref, o_ref):
        o_ref[...] = (
            jnp.dot(x_ref[...], w_ref[...], preferred_element_type=jnp.float32)
            + b_ref[...]
        )

    return pl.pallas_call(
        body,
        grid=(n // BM,),
        in_specs=[
            pl.BlockSpec((BM, HIDDEN), lambda i: (i, 0)),
            pl.BlockSpec((HIDDEN, EMSIZE), lambda i: (0, 0)),
            pl.BlockSpec((1, EMSIZE), lambda i: (0, 0)),
        ],
        out_specs=pl.BlockSpec((BM, EMSIZE), lambda i: (i, 0)),
        out_shape=jax.ShapeDtypeStruct((n, EMSIZE), jnp.float32),
    )(x, w, b.reshape(1, EMSIZE))


def kernel(item_embeds, emb_table, W_proj, b_proj):
    batch, hist = item_embeds.shape
    idx = item_embeds.T.reshape(-1)
    rows = _sc_gather(emb_table, idx)
    out = _tc_project(rows, W_proj, b_proj)
    return out.reshape(hist, batch, EMSIZE).transpose(1, 0, 2)
